# baseline (speedup 1.0000x reference)
import functools

import jax
import jax.numpy as jnp
from jax import lax
from jax.experimental import pallas as pl
from jax.experimental.pallas import tpu as pltpu
from jax.experimental.pallas import tpu_sc as plsc
from jax.scipy.linalg import block_diag

N = 10000
E = 160000
CIN = 128
H = 128
T = 8
TD = 16
L = 4
M = 3
ED = 4
HID = TD * 2
B = 512

F32 = jnp.float32

NW = 32
EP = 163840
EPW = EP // NW
CH = 128
NCH = EPW // CH
NP = 10240
RPT = NP // 16
ZR = 40

NB = 1000
EB = 1024


def _silu(v):
    return v * jax.nn.sigmoid(v)


def _k1_body(x_ref, w_ref, b_ref, o_ref):
    o_ref[...] = _silu(jnp.dot(x_ref[...], w_ref[...],
                               preferred_element_type=F32) + b_ref[...])


def _k1(x, W1, b1r):
    return pl.pallas_call(
        _k1_body,
        grid=(N // NB,),
        in_specs=[
            pl.BlockSpec((NB, CIN), lambda i: (i, 0)),
            pl.BlockSpec((CIN, H), lambda i: (0, 0)),
            pl.BlockSpec((1, H), lambda i: (0, 0)),
        ],
        out_specs=pl.BlockSpec((NB, H), lambda i: (i, 0)),
        out_shape=jax.ShapeDtypeStruct((N, H), F32),
    )(x, W1, b1r)


def _k2_body(h0_hbm, src_hbm, out_hbm, idx_v, rows0, rows1, sem0, sem1):
    cid = lax.axis_index("c")
    sid = lax.axis_index("s")
    wid = cid * 16 + sid
    pltpu.sync_copy(src_hbm.at[wid], idx_v)
    base = wid * EPW

    bufs = (rows0, rows1)
    sems = (sem0, sem1)
    for c in range(NCH):
        p = c % 2
        pltpu.async_copy(h0_hbm.at[idx_v.at[c]], bufs[p], sems[p]).wait()
        pltpu.sync_copy(bufs[p], out_hbm.at[pl.ds(base + c * CH, CH)])


def _k2(h0, src_r):
    mesh = plsc.VectorSubcoreMesh(core_axis_name="c", subcore_axis_name="s", num_cores=2, num_subcores=16)
    f = functools.partial(
        pl.kernel,
        out_type=jax.ShapeDtypeStruct((EP, H), F32),
        mesh=mesh,
        scratch_types=[
            pltpu.VMEM((NCH, CH), jnp.int32),
            pltpu.VMEM((CH, H), F32),
            pltpu.VMEM((CH, H), F32),
            pltpu.SemaphoreType.DMA,
            pltpu.SemaphoreType.DMA,
        ],
    )(_k2_body)
    return f(h0, src_r)


def _k3_body(ea_ref, xj_ref, a1_ref, b1_ref, a2_ref, b2_ref, rep_ref,
             red_ref, msg_ref):
    ea = ea_ref[...]
    xj = xj_ref[...]
    hid = _silu(jnp.dot(ea, a1_ref[...], preferred_element_type=F32)
                + b1_ref[...])
    rep = rep_ref[...]
    red = red_ref[...]
    parts = []
    for t in range(T):
        ew = jnp.dot(hid[:, t * HID:(t + 1) * HID], a2_ref[t],
                     preferred_element_type=F32) + b2_ref[t]
        xr = jnp.dot(xj[:, t * TD:(t + 1) * TD], rep,
                     preferred_element_type=F32)
        parts.append(jnp.dot(xr * ew, red, preferred_element_type=F32))
    msg_ref[...] = jnp.concatenate(parts, axis=1)


def _k3(ea, xj, a1c, b1c, nnA2, nnb2r, rep, red):
    return pl.pallas_call(
        _k3_body,
        grid=(EP // EB,),
        in_specs=[
            pl.BlockSpec((EB, ED), lambda i: (i, 0)),
            pl.BlockSpec((EB, H), lambda i: (i, 0)),
            pl.BlockSpec((ED, T * HID), lambda i: (0, 0)),
            pl.BlockSpec((1, T * HID), lambda i: (0, 0)),
            pl.BlockSpec((T, HID, TD * TD), lambda i: (0, 0, 0)),
            pl.BlockSpec((T, 1, TD * TD), lambda i: (0, 0, 0)),
            pl.BlockSpec((TD, TD * TD), lambda i: (0, 0)),
            pl.BlockSpec((TD * TD, TD), lambda i: (0, 0)),
        ],
        out_specs=pl.BlockSpec((EB, H), lambda i: (i, 0)),
        out_shape=jax.ShapeDtypeStruct((EP, H), F32),
    )(ea, xj, a1c, b1c, nnA2, nnb2r, rep, red)


def _k4_body(msg_hbm, dst_hbm, out_hbm, idx_v, rows0, rows1, zbuf, acc_sh,
             sem0, sem1):
    cid = lax.axis_index("c")
    sid = lax.axis_index("s")
    wid = cid * 16 + sid

    zero = jnp.zeros((16,), F32)
    for r in range(ZR):
        for g in range(H // 16):
            zbuf[r, pl.ds(g * 16, 16)] = zero
    for j in range(RPT // ZR):
        pltpu.sync_copy(zbuf, acc_sh.at[pl.ds(sid * RPT + j * ZR, ZR)])
    plsc.subcore_barrier()

    pltpu.sync_copy(dst_hbm.at[wid], idx_v)
    base = wid * EPW

    bufs = (rows0, rows1)
    sems = (sem0, sem1)
    d = pltpu.async_copy(msg_hbm.at[pl.ds(base, CH)], bufs[0], sems[0])
    for c in range(NCH):
        if c + 1 < NCH:
            d_next = pltpu.async_copy(
                msg_hbm.at[pl.ds(base + (c + 1) * CH, CH)],
                bufs[(c + 1) % 2], sems[(c + 1) % 2])
        d.wait()
        pltpu.sync_copy(bufs[c % 2], acc_sh.at[idx_v.at[c]], add=True)
        if c + 1 < NCH:
            d = d_next
    plsc.subcore_barrier()

    pltpu.sync_copy(acc_sh.at[pl.ds(sid * RPT, RPT)],
                    out_hbm.at[pl.ds(cid * NP + sid * RPT, RPT)])


def _k4(msg, dst_r):
    mesh = plsc.VectorSubcoreMesh(core_axis_name="c", subcore_axis_name="s", num_cores=2, num_subcores=16)
    f = functools.partial(
        pl.kernel,
        out_type=jax.ShapeDtypeStruct((2 * NP, H), F32),
        mesh=mesh,
        scratch_types=[
            pltpu.VMEM((NCH, CH), jnp.int32),
            pltpu.VMEM((CH, H), F32),
            pltpu.VMEM((CH, H), F32),
            pltpu.VMEM((ZR, H), F32),
            pltpu.VMEM_SHARED((NP, H), F32),
            pltpu.SemaphoreType.DMA,
            pltpu.SemaphoreType.DMA,
        ],
    )(_k4_body)
    return f(msg, dst_r)


def _k5_body(h0_ref, p0_ref, p1_ref, root_ref, cb_ref,
             wir_ref, wiz_ref, win_ref, whr_ref, whz_ref, whn_ref,
             bir_ref, biz_ref, bin_ref, bhr_ref, bhz_ref, bhn_ref,
             mw1_ref, mb1_ref, mw2_ref, mb2_ref, o_ref):
    xt = h0_ref[...]
    aggr = p0_ref[...] + p1_ref[...]
    conv = _silu(aggr + jnp.dot(xt, root_ref[...],
                                preferred_element_type=F32) + cb_ref[...])
    gir = jnp.dot(conv, wir_ref[...], preferred_element_type=F32) + bir_ref[...]
    giz = jnp.dot(conv, wiz_ref[...], preferred_element_type=F32) + biz_ref[...]
    gin = jnp.dot(conv, win_ref[...], preferred_element_type=F32) + bin_ref[...]
    h = xt
    for _ in range(L):
        ghr = jnp.dot(h, whr_ref[...], preferred_element_type=F32) + bhr_ref[...]
        ghz = jnp.dot(h, whz_ref[...], preferred_element_type=F32) + bhz_ref[...]
        ghn = jnp.dot(h, whn_ref[...], preferred_element_type=F32) + bhn_ref[...]
        r = jax.nn.sigmoid(gir + ghr)
        z = jax.nn.sigmoid(giz + ghz)
        n = jnp.tanh(gin + r * ghn)
        h = (1.0 - z) * n + z * h
    mixed = jnp.dot(_silu(jnp.dot(h, mw1_ref[...],
                                  preferred_element_type=F32) + mb1_ref[...]),
                    mw2_ref[...], preferred_element_type=F32) + mb2_ref[...]
    o_ref[...] = mixed


def _k5(h0, p0, p1, weights):
    blocked = pl.BlockSpec((NB, H), lambda i: (i, 0))
    wspec = [pl.BlockSpec(w.shape, lambda i: (0, 0)) for w in weights]
    return pl.pallas_call(
        _k5_body,
        grid=(N // NB,),
        in_specs=[blocked, blocked, blocked] + wspec,
        out_specs=blocked,
        out_shape=jax.ShapeDtypeStruct((N, H), F32),
    )(h0, p0, p1, *weights)


def _k6_body(mx_ref, bc_ref, br_ref, lwi_ref, lwh_ref, lb_ref,
             ow1_ref, ob1_ref, ow2_ref, ob2_ref, o_ref, e_scr, a_scr):
    CN = N // NB
    hl = jnp.zeros((B, H), F32)
    cl = jnp.zeros((B, H), F32)
    q_star = jnp.zeros((B, 2 * H), F32)
    neg = jnp.float32(-1e30)

    for _ in range(M):
        g = (jnp.dot(q_star, lwi_ref[...], preferred_element_type=F32)
             + jnp.dot(hl, lwh_ref[...], preferred_element_type=F32)
             + lb_ref[...])
        ig = jax.nn.sigmoid(g[:, :H])
        fg = jax.nn.sigmoid(g[:, H:2 * H])
        gg = jnp.tanh(g[:, 2 * H:3 * H])
        og = jax.nn.sigmoid(g[:, 3 * H:])
        cl = fg * cl + ig * gg
        hl = og * jnp.tanh(cl)
        q = hl

        em = jnp.full((1, B), neg, F32)
        for c in range(CN):
            mxc = mx_ref[pl.ds(c * NB, NB), :]
            bcc = bc_ref[pl.ds(c * NB, NB), :]
            oh = (lax.broadcasted_iota(jnp.int32, (NB, B), 1) == bcc)
            ohf = oh.astype(F32)
            qb = jnp.dot(ohf, q, preferred_element_type=F32)
            e_c = jnp.sum(mxc * qb, axis=1, keepdims=True)
            e_scr[pl.ds(c * NB, NB), :] = e_c
            em = jnp.maximum(em, jnp.max(jnp.where(oh, e_c, neg), axis=0,
                                         keepdims=True))

        den = jnp.zeros((1, B), F32)
        for c in range(CN):
            bcc = bc_ref[pl.ds(c * NB, NB), :]
            oh = (lax.broadcasted_iota(jnp.int32, (NB, B), 1) == bcc)
            ohf = oh.astype(F32)
            e_c = e_scr[pl.ds(c * NB, NB), :]
            emg = jnp.sum(ohf * em, axis=1, keepdims=True)
            a_c = jnp.exp(e_c - emg)
            a_scr[pl.ds(c * NB, NB), :] = a_c
            den = den + jnp.sum(ohf * a_c, axis=0, keepdims=True)

        r2 = jnp.zeros((B, H), F32)
        for c in range(CN):
            mxc = mx_ref[pl.ds(c * NB, NB), :]
            bcc = bc_ref[pl.ds(c * NB, NB), :]
            brc = br_ref[c]
            oh = (lax.broadcasted_iota(jnp.int32, (NB, B), 1) == bcc)
            ohf = oh.astype(F32)
            ohT = (lax.broadcasted_iota(jnp.int32, (B, NB), 0) == brc)
            a_c = a_scr[pl.ds(c * NB, NB), :]
            deng = jnp.sum(ohf * den, axis=1, keepdims=True)
            an = a_c / deng
            r2 = r2 + jnp.dot(ohT.astype(F32), an * mxc,
                              preferred_element_type=F32)
        q_star = jnp.concatenate([q, r2], axis=1)

    o = (jnp.dot(_silu(jnp.dot(q_star, ow1_ref[...],
                               preferred_element_type=F32) + ob1_ref[...]),
                 ow2_ref[...], preferred_element_type=F32) + ob2_ref[...])
    o_ref[...] = o


def _k6(mixed, batch_col, batch_row, lwi, lwh, lbr, ow1, ob1, ow2, ob2):
    return pl.pallas_call(
        _k6_body,
        out_shape=jax.ShapeDtypeStruct((B, 1), F32),
        scratch_shapes=[
            pltpu.VMEM((N, 1), F32),
            pltpu.VMEM((N, 1), F32),
        ],
    )(mixed, batch_col, batch_row, lwi, lwh, lbr, ow1, ob1, ow2, ob2)


def kernel(x, edge_attr, W1, b1, nnA1, nnb1, nnA2, nnb2, root, conv_b,
           gru_Wih, gru_Whh, gru_bih, gru_bhh, mix_W1, mix_b1, mix_W2,
           mix_b2, lstm_Wih, lstm_Whh, lstm_bih, lstm_bhh, out_W1, out_b1,
           out_W2, out_b2, edge_index, batch):
    b1r = b1.reshape(1, H)
    a1c = nnA1.transpose(1, 0, 2).reshape(ED, T * HID)
    b1c = nnb1.reshape(1, T * HID)
    nnb2r = nnb2.reshape(T, 1, TD * TD)
    rep = jnp.kron(jnp.eye(TD, dtype=F32), jnp.ones((1, TD), F32))
    red = jnp.kron(jnp.ones((TD, 1), F32), jnp.eye(TD, dtype=F32))

    rootbd = block_diag(*[root[t] for t in range(T)])
    cbr = conv_b.reshape(1, H)
    wir = block_diag(*[gru_Wih[t][:, :TD] for t in range(T)])
    wiz = block_diag(*[gru_Wih[t][:, TD:2 * TD] for t in range(T)])
    win = block_diag(*[gru_Wih[t][:, 2 * TD:] for t in range(T)])
    whr = block_diag(*[gru_Whh[t][:, :TD] for t in range(T)])
    whz = block_diag(*[gru_Whh[t][:, TD:2 * TD] for t in range(T)])
    whn = block_diag(*[gru_Whh[t][:, 2 * TD:] for t in range(T)])
    bir = gru_bih[:, :TD].reshape(1, H)
    biz = gru_bih[:, TD:2 * TD].reshape(1, H)
    bin_ = gru_bih[:, 2 * TD:].reshape(1, H)
    bhr = gru_bhh[:, :TD].reshape(1, H)
    bhz = gru_bhh[:, TD:2 * TD].reshape(1, H)
    bhn = gru_bhh[:, 2 * TD:].reshape(1, H)
    mb1 = mix_b1.reshape(1, H)
    mb2 = mix_b2.reshape(1, H)

    lbr = (lstm_bih + lstm_bhh).reshape(1, 4 * H)
    ob1 = out_b1.reshape(1, H)
    ob2 = out_b2.reshape(1, 1)

    pad = EP - E
    src_r = jnp.concatenate(
        [edge_index[0], jnp.zeros((pad,), jnp.int32)]).reshape(NW, NCH, CH)
    dst_r = jnp.concatenate(
        [edge_index[1], jnp.full((pad,), NP - 1, jnp.int32)]).reshape(
            NW, NCH, CH)
    ea_pad = jnp.concatenate(
        [edge_attr, jnp.zeros((pad, ED), F32)], axis=0)
    batch_col = batch.reshape(N, 1)
    batch_row = batch.reshape(N // NB, 1, NB)

    h0 = _k1(x, W1, b1r)
    xj = _k2(h0, src_r)
    msg = _k3(ea_pad, xj, a1c, b1c, nnA2, nnb2r, rep, red)
    parts = _k4(msg, dst_r)
    k5w = [rootbd, cbr, wir, wiz, win, whr, whz, whn,
           bir, biz, bin_, bhr, bhz, bhn, mix_W1, mb1, mix_W2, mb2]
    mixed = _k5(h0, parts[:N], parts[NP:NP + N], k5w)
    o = _k6(mixed, batch_col, batch_row, lstm_Wih, lstm_Whh, lbr,
            out_W1, ob1, out_W2, ob2)
    return o[:, 0]


# speedup vs baseline: 1.2823x; 1.2823x over previous
import functools

import jax
import jax.numpy as jnp
from jax import lax
from jax.experimental import pallas as pl
from jax.experimental.pallas import tpu as pltpu
from jax.experimental.pallas import tpu_sc as plsc
from jax.scipy.linalg import block_diag

N = 10000
E = 160000
CIN = 128
H = 128
T = 8
TD = 16
L = 4
M = 3
ED = 4
HID = TD * 2
B = 512

F32 = jnp.float32

NW = 32
EP = 160000
EPW = EP // NW
CH = 40
NCH = EPW // CH
NP = 10240
RPT = NP // 16
ZR = 40

NB = 1000
EB = 1000


def _silu(v):
    return v * jax.nn.sigmoid(v)


def _k1_body(x_ref, w_ref, b_ref, o_ref):
    o_ref[...] = _silu(jnp.dot(x_ref[...], w_ref[...],
                               preferred_element_type=F32) + b_ref[...])


def _k1(x, W1, b1r):
    return pl.pallas_call(
        _k1_body,
        grid=(N // NB,),
        in_specs=[
            pl.BlockSpec((NB, CIN), lambda i: (i, 0)),
            pl.BlockSpec((CIN, H), lambda i: (0, 0)),
            pl.BlockSpec((1, H), lambda i: (0, 0)),
        ],
        out_specs=pl.BlockSpec((NB, H), lambda i: (i, 0)),
        out_shape=jax.ShapeDtypeStruct((N, H), F32),
    )(x, W1, b1r)


def _k2_body(h0_hbm, src_hbm, out_hbm, idx_v, rows0, rows1, sem0, sem1):
    cid = lax.axis_index("c")
    sid = lax.axis_index("s")
    wid = cid * 16 + sid
    pltpu.sync_copy(src_hbm.at[wid], idx_v)
    base = wid * EPW

    bufs = (rows0, rows1)
    sems = (sem0, sem1)
    for c in range(NCH):
        p = c % 2
        pltpu.async_copy(h0_hbm.at[idx_v.at[c]], bufs[p], sems[p]).wait()
        pltpu.sync_copy(bufs[p], out_hbm.at[pl.ds(base + c * CH, CH)])


def _k2(h0, src_r):
    mesh = plsc.VectorSubcoreMesh(core_axis_name="c", subcore_axis_name="s", num_cores=2, num_subcores=16)
    f = functools.partial(
        pl.kernel,
        out_type=jax.ShapeDtypeStruct((EP, H), F32),
        mesh=mesh,
        scratch_types=[
            pltpu.VMEM((NCH, CH), jnp.int32),
            pltpu.VMEM((CH, H), F32),
            pltpu.VMEM((CH, H), F32),
            pltpu.SemaphoreType.DMA,
            pltpu.SemaphoreType.DMA,
        ],
    )(_k2_body)
    return f(h0, src_r)


def _k3_body(ea_ref, xj_ref, a1_ref, b1_ref, a2_ref, b2_ref, red_ref,
             msg_ref):
    ea = ea_ref[...]
    xj = xj_ref[...]
    hid = _silu(jnp.dot(ea, a1_ref[...], preferred_element_type=F32)
                + b1_ref[...])
    red = red_ref[...]
    parts = []
    for t in range(T):
        ew = jnp.dot(hid[:, t * HID:(t + 1) * HID], a2_ref[t],
                     preferred_element_type=F32) + b2_ref[t]
        xt = xj[:, t * TD:(t + 1) * TD]
        xr = jnp.concatenate([xt] * TD, axis=1)
        parts.append(jnp.dot(xr * ew, red, preferred_element_type=F32))
    msg_ref[...] = jnp.concatenate(parts, axis=1)


def _k3(ea, xj, a1c, b1c, nnA2p, nnb2p, red):
    return pl.pallas_call(
        _k3_body,
        grid=(EP // EB,),
        in_specs=[
            pl.BlockSpec((EB, ED), lambda i: (i, 0)),
            pl.BlockSpec((EB, H), lambda i: (i, 0)),
            pl.BlockSpec((ED, T * HID), lambda i: (0, 0)),
            pl.BlockSpec((1, T * HID), lambda i: (0, 0)),
            pl.BlockSpec((T, HID, TD * TD), lambda i: (0, 0, 0)),
            pl.BlockSpec((T, 1, TD * TD), lambda i: (0, 0, 0)),
            pl.BlockSpec((TD * TD, TD), lambda i: (0, 0)),
        ],
        out_specs=pl.BlockSpec((EB, H), lambda i: (i, 0)),
        out_shape=jax.ShapeDtypeStruct((EP, H), F32),
    )(ea, xj, a1c, b1c, nnA2p, nnb2p, red)


def _k4_body(msg_hbm, dst_hbm, out_hbm, idx_v, rows0, rows1, zbuf, acc_sh,
             sem0, sem1):
    cid = lax.axis_index("c")
    sid = lax.axis_index("s")
    wid = cid * 16 + sid

    zero = jnp.zeros((16,), F32)
    for r in range(ZR):
        for g in range(H // 16):
            zbuf[r, pl.ds(g * 16, 16)] = zero
    for j in range(RPT // ZR):
        pltpu.sync_copy(zbuf, acc_sh.at[pl.ds(sid * RPT + j * ZR, ZR)])
    plsc.subcore_barrier()

    pltpu.sync_copy(dst_hbm.at[wid], idx_v)
    base = wid * EPW

    bufs = (rows0, rows1)
    sems = (sem0, sem1)
    d = pltpu.async_copy(msg_hbm.at[pl.ds(base, CH)], bufs[0], sems[0])
    for c in range(NCH):
        if c + 1 < NCH:
            d_next = pltpu.async_copy(
                msg_hbm.at[pl.ds(base + (c + 1) * CH, CH)],
                bufs[(c + 1) % 2], sems[(c + 1) % 2])
        d.wait()
        pltpu.sync_copy(bufs[c % 2], acc_sh.at[idx_v.at[c]], add=True)
        if c + 1 < NCH:
            d = d_next
    plsc.subcore_barrier()

    pltpu.sync_copy(acc_sh.at[pl.ds(sid * RPT, RPT)],
                    out_hbm.at[pl.ds(cid * NP + sid * RPT, RPT)])


def _k4(msg, dst_r):
    mesh = plsc.VectorSubcoreMesh(core_axis_name="c", subcore_axis_name="s", num_cores=2, num_subcores=16)
    f = functools.partial(
        pl.kernel,
        out_type=jax.ShapeDtypeStruct((2 * NP, H), F32),
        mesh=mesh,
        scratch_types=[
            pltpu.VMEM((NCH, CH), jnp.int32),
            pltpu.VMEM((CH, H), F32),
            pltpu.VMEM((CH, H), F32),
            pltpu.VMEM((ZR, H), F32),
            pltpu.VMEM_SHARED((NP, H), F32),
            pltpu.SemaphoreType.DMA,
            pltpu.SemaphoreType.DMA,
        ],
    )(_k4_body)
    return f(msg, dst_r)


def _k5_body(h0_ref, p0_ref, p1_ref, root_ref, cb_ref,
             wir_ref, wiz_ref, win_ref, whr_ref, whz_ref, whn_ref,
             bir_ref, biz_ref, bin_ref, bhr_ref, bhz_ref, bhn_ref,
             mw1_ref, mb1_ref, mw2_ref, mb2_ref, o_ref):
    xt = h0_ref[...]
    aggr = p0_ref[...] + p1_ref[...]
    conv = _silu(aggr + jnp.dot(xt, root_ref[...],
                                preferred_element_type=F32) + cb_ref[...])
    gir = jnp.dot(conv, wir_ref[...], preferred_element_type=F32) + bir_ref[...]
    giz = jnp.dot(conv, wiz_ref[...], preferred_element_type=F32) + biz_ref[...]
    gin = jnp.dot(conv, win_ref[...], preferred_element_type=F32) + bin_ref[...]
    h = xt
    for _ in range(L):
        ghr = jnp.dot(h, whr_ref[...], preferred_element_type=F32) + bhr_ref[...]
        ghz = jnp.dot(h, whz_ref[...], preferred_element_type=F32) + bhz_ref[...]
        ghn = jnp.dot(h, whn_ref[...], preferred_element_type=F32) + bhn_ref[...]
        r = jax.nn.sigmoid(gir + ghr)
        z = jax.nn.sigmoid(giz + ghz)
        n = jnp.tanh(gin + r * ghn)
        h = (1.0 - z) * n + z * h
    mixed = jnp.dot(_silu(jnp.dot(h, mw1_ref[...],
                                  preferred_element_type=F32) + mb1_ref[...]),
                    mw2_ref[...], preferred_element_type=F32) + mb2_ref[...]
    o_ref[...] = mixed


def _k5(h0, p0, p1, weights):
    blocked = pl.BlockSpec((NB, H), lambda i: (i, 0))
    wspec = [pl.BlockSpec(w.shape, lambda i: (0, 0)) for w in weights]
    return pl.pallas_call(
        _k5_body,
        grid=(N // NB,),
        in_specs=[blocked, blocked, blocked] + wspec,
        out_specs=blocked,
        out_shape=jax.ShapeDtypeStruct((N, H), F32),
    )(h0, p0, p1, *weights)


def _k6_body(mx_ref, bc_ref, br_ref, lwi_ref, lwh_ref, lb_ref,
             ow1_ref, ob1_ref, ow2_ref, ob2_ref, o_ref, e_scr, a_scr):
    CN = N // NB
    hl = jnp.zeros((B, H), F32)
    cl = jnp.zeros((B, H), F32)
    q_star = jnp.zeros((B, 2 * H), F32)
    neg = jnp.float32(-1e30)

    for _ in range(M):
        g = (jnp.dot(q_star, lwi_ref[...], preferred_element_type=F32)
             + jnp.dot(hl, lwh_ref[...], preferred_element_type=F32)
             + lb_ref[...])
        ig = jax.nn.sigmoid(g[:, :H])
        fg = jax.nn.sigmoid(g[:, H:2 * H])
        gg = jnp.tanh(g[:, 2 * H:3 * H])
        og = jax.nn.sigmoid(g[:, 3 * H:])
        cl = fg * cl + ig * gg
        hl = og * jnp.tanh(cl)
        q = hl

        em = jnp.full((1, B), neg, F32)
        for c in range(CN):
            mxc = mx_ref[pl.ds(c * NB, NB), :]
            bcc = bc_ref[pl.ds(c * NB, NB), :]
            oh = (lax.broadcasted_iota(jnp.int32, (NB, B), 1) == bcc)
            ohf = oh.astype(F32)
            qb = jnp.dot(ohf, q, preferred_element_type=F32)
            e_c = jnp.sum(mxc * qb, axis=1, keepdims=True)
            e_scr[pl.ds(c * NB, NB), :] = e_c
            em = jnp.maximum(em, jnp.max(jnp.where(oh, e_c, neg), axis=0,
                                         keepdims=True))

        den = jnp.zeros((1, B), F32)
        for c in range(CN):
            bcc = bc_ref[pl.ds(c * NB, NB), :]
            oh = (lax.broadcasted_iota(jnp.int32, (NB, B), 1) == bcc)
            ohf = oh.astype(F32)
            e_c = e_scr[pl.ds(c * NB, NB), :]
            emg = jnp.sum(ohf * em, axis=1, keepdims=True)
            a_c = jnp.exp(e_c - emg)
            a_scr[pl.ds(c * NB, NB), :] = a_c
            den = den + jnp.sum(ohf * a_c, axis=0, keepdims=True)

        r2 = jnp.zeros((B, H), F32)
        for c in range(CN):
            mxc = mx_ref[pl.ds(c * NB, NB), :]
            bcc = bc_ref[pl.ds(c * NB, NB), :]
            brc = br_ref[c]
            oh = (lax.broadcasted_iota(jnp.int32, (NB, B), 1) == bcc)
            ohf = oh.astype(F32)
            ohT = (lax.broadcasted_iota(jnp.int32, (B, NB), 0) == brc)
            a_c = a_scr[pl.ds(c * NB, NB), :]
            deng = jnp.sum(ohf * den, axis=1, keepdims=True)
            an = a_c / deng
            r2 = r2 + jnp.dot(ohT.astype(F32), an * mxc,
                              preferred_element_type=F32)
        q_star = jnp.concatenate([q, r2], axis=1)

    o = (jnp.dot(_silu(jnp.dot(q_star, ow1_ref[...],
                               preferred_element_type=F32) + ob1_ref[...]),
                 ow2_ref[...], preferred_element_type=F32) + ob2_ref[...])
    o_ref[...] = o


def _k6(mixed, batch_col, batch_row, lwi, lwh, lbr, ow1, ob1, ow2, ob2):
    return pl.pallas_call(
        _k6_body,
        out_shape=jax.ShapeDtypeStruct((B, 1), F32),
        scratch_shapes=[
            pltpu.VMEM((N, 1), F32),
            pltpu.VMEM((N, 1), F32),
        ],
    )(mixed, batch_col, batch_row, lwi, lwh, lbr, ow1, ob1, ow2, ob2)


def kernel(x, edge_attr, W1, b1, nnA1, nnb1, nnA2, nnb2, root, conv_b,
           gru_Wih, gru_Whh, gru_bih, gru_bhh, mix_W1, mix_b1, mix_W2,
           mix_b2, lstm_Wih, lstm_Whh, lstm_bih, lstm_bhh, out_W1, out_b1,
           out_W2, out_b2, edge_index, batch):
    b1r = b1.reshape(1, H)
    a1c = nnA1.transpose(1, 0, 2).reshape(ED, T * HID)
    b1c = nnb1.reshape(1, T * HID)
    nnA2p = nnA2.reshape(T, HID, TD, TD).transpose(0, 1, 3, 2).reshape(
        T, HID, TD * TD)
    nnb2p = nnb2.reshape(T, TD, TD).transpose(0, 2, 1).reshape(
        T, 1, TD * TD)
    red = jnp.kron(jnp.eye(TD, dtype=F32), jnp.ones((TD, 1), F32))

    rootbd = block_diag(*[root[t] for t in range(T)])
    cbr = conv_b.reshape(1, H)
    wir = block_diag(*[gru_Wih[t][:, :TD] for t in range(T)])
    wiz = block_diag(*[gru_Wih[t][:, TD:2 * TD] for t in range(T)])
    win = block_diag(*[gru_Wih[t][:, 2 * TD:] for t in range(T)])
    whr = block_diag(*[gru_Whh[t][:, :TD] for t in range(T)])
    whz = block_diag(*[gru_Whh[t][:, TD:2 * TD] for t in range(T)])
    whn = block_diag(*[gru_Whh[t][:, 2 * TD:] for t in range(T)])
    bir = gru_bih[:, :TD].reshape(1, H)
    biz = gru_bih[:, TD:2 * TD].reshape(1, H)
    bin_ = gru_bih[:, 2 * TD:].reshape(1, H)
    bhr = gru_bhh[:, :TD].reshape(1, H)
    bhz = gru_bhh[:, TD:2 * TD].reshape(1, H)
    bhn = gru_bhh[:, 2 * TD:].reshape(1, H)
    mb1 = mix_b1.reshape(1, H)
    mb2 = mix_b2.reshape(1, H)

    lbr = (lstm_bih + lstm_bhh).reshape(1, 4 * H)
    ob1 = out_b1.reshape(1, H)
    ob2 = out_b2.reshape(1, 1)

    pad = EP - E
    src_r = jnp.concatenate(
        [edge_index[0], jnp.zeros((pad,), jnp.int32)]).reshape(NW, NCH, CH)
    dst_r = jnp.concatenate(
        [edge_index[1], jnp.full((pad,), NP - 1, jnp.int32)]).reshape(
            NW, NCH, CH)
    ea_pad = jnp.concatenate(
        [edge_attr, jnp.zeros((pad, ED), F32)], axis=0)
    batch_col = batch.reshape(N, 1)
    batch_row = batch.reshape(N // NB, 1, NB)

    h0 = _k1(x, W1, b1r)
    xj = _k2(h0, src_r)
    msg = _k3(ea_pad, xj, a1c, b1c, nnA2p, nnb2p, red)
    parts = _k4(msg, dst_r)
    k5w = [rootbd, cbr, wir, wiz, win, whr, whz, whn,
           bir, biz, bin_, bhr, bhz, bhn, mix_W1, mb1, mix_W2, mb2]
    mixed = _k5(h0, parts[:N], parts[NP:NP + N], k5w)
    o = _k6(mixed, batch_col, batch_row, lstm_Wih, lstm_Whh, lbr,
            out_W1, ob1, out_W2, ob2)
    return o[:, 0]


# speedup vs baseline: 1.3787x; 1.0752x over previous
import functools

import jax
import jax.numpy as jnp
from jax import lax
from jax.experimental import pallas as pl
from jax.experimental.pallas import tpu as pltpu
from jax.experimental.pallas import tpu_sc as plsc
from jax.scipy.linalg import block_diag

N = 10000
E = 160000
CIN = 128
H = 128
T = 8
TD = 16
L = 4
M = 3
ED = 4
HID = TD * 2
B = 512

F32 = jnp.float32

NW = 32
EP = 160000
EPW = EP // NW
CH = 40
NCH = EPW // CH
NP = 10240
RPT = NP // 16
ZR = 40

NB = 1000
EB = 1000


def _silu(v):
    return v * jax.nn.sigmoid(v)


def _k1_body(x_ref, w_ref, b_ref, o_ref):
    o_ref[...] = _silu(jnp.dot(x_ref[...], w_ref[...],
                               preferred_element_type=F32) + b_ref[...])


def _k1(x, W1, b1r):
    return pl.pallas_call(
        _k1_body,
        grid=(N // NB,),
        in_specs=[
            pl.BlockSpec((NB, CIN), lambda i: (i, 0)),
            pl.BlockSpec((CIN, H), lambda i: (0, 0)),
            pl.BlockSpec((1, H), lambda i: (0, 0)),
        ],
        out_specs=pl.BlockSpec((NB, H), lambda i: (i, 0)),
        out_shape=jax.ShapeDtypeStruct((N, H), F32),
    )(x, W1, b1r)


def _k2_body(h0_hbm, src_hbm, out_hbm, idx_v, rows0, rows1, sem0, sem1):
    cid = lax.axis_index("c")
    sid = lax.axis_index("s")
    wid = cid * 16 + sid
    pltpu.sync_copy(src_hbm.at[wid], idx_v)
    base = wid * EPW

    bufs = (rows0, rows1)
    sems = (sem0, sem1)
    d = pltpu.async_copy(h0_hbm.at[idx_v.at[0]], bufs[0], sems[0])
    for c in range(NCH):
        if c + 1 < NCH:
            d_next = pltpu.async_copy(h0_hbm.at[idx_v.at[c + 1]],
                                      bufs[(c + 1) % 2], sems[(c + 1) % 2])
        d.wait()
        pltpu.sync_copy(bufs[c % 2], out_hbm.at[pl.ds(base + c * CH, CH)])
        if c + 1 < NCH:
            d = d_next


def _k2(h0, src_r):
    mesh = plsc.VectorSubcoreMesh(core_axis_name="c", subcore_axis_name="s", num_cores=2, num_subcores=16)
    f = functools.partial(
        pl.kernel,
        out_type=jax.ShapeDtypeStruct((EP, H), F32),
        mesh=mesh,
        scratch_types=[
            pltpu.VMEM((NCH, CH), jnp.int32),
            pltpu.VMEM((CH, H), F32),
            pltpu.VMEM((CH, H), F32),
            pltpu.SemaphoreType.DMA,
            pltpu.SemaphoreType.DMA,
        ],
    )(_k2_body)
    return f(h0, src_r)


def _k3_body(ea_ref, xj_ref, a1_ref, b1_ref, a2_ref, b2_ref, red_ref,
             msg_ref):
    ea = ea_ref[...]
    xj = xj_ref[...]
    hid = _silu(jnp.dot(ea, a1_ref[...], preferred_element_type=F32)
                + b1_ref[...])
    red = red_ref[...]
    parts = []
    for t in range(T):
        ew = jnp.dot(hid[:, t * HID:(t + 1) * HID], a2_ref[t],
                     preferred_element_type=F32) + b2_ref[t]
        xt = xj[:, t * TD:(t + 1) * TD]
        xr = jnp.concatenate([xt] * TD, axis=1)
        parts.append(jnp.dot(xr * ew, red, preferred_element_type=F32))
    msg_ref[...] = jnp.concatenate(parts, axis=1)


def _k3(ea, xj, a1c, b1c, nnA2p, nnb2p, red):
    return pl.pallas_call(
        _k3_body,
        grid=(EP // EB,),
        in_specs=[
            pl.BlockSpec((EB, ED), lambda i: (i, 0)),
            pl.BlockSpec((EB, H), lambda i: (i, 0)),
            pl.BlockSpec((ED, T * HID), lambda i: (0, 0)),
            pl.BlockSpec((1, T * HID), lambda i: (0, 0)),
            pl.BlockSpec((T, HID, TD * TD), lambda i: (0, 0, 0)),
            pl.BlockSpec((T, 1, TD * TD), lambda i: (0, 0, 0)),
            pl.BlockSpec((TD * TD, TD), lambda i: (0, 0)),
        ],
        out_specs=pl.BlockSpec((EB, H), lambda i: (i, 0)),
        out_shape=jax.ShapeDtypeStruct((EP, H), F32),
    )(ea, xj, a1c, b1c, nnA2p, nnb2p, red)


def _k4_body(msg_hbm, dst_hbm, out_hbm, idx_v, rows0, rows1, zbuf, acc_sh,
             sem0, sem1):
    cid = lax.axis_index("c")
    sid = lax.axis_index("s")
    wid = cid * 16 + sid

    zero = jnp.zeros((16,), F32)
    for r in range(ZR):
        for g in range(H // 16):
            zbuf[r, pl.ds(g * 16, 16)] = zero
    for j in range(RPT // ZR):
        pltpu.sync_copy(zbuf, acc_sh.at[pl.ds(sid * RPT + j * ZR, ZR)])
    plsc.subcore_barrier()

    pltpu.sync_copy(dst_hbm.at[wid], idx_v)
    base = wid * EPW

    bufs = (rows0, rows1)
    sems = (sem0, sem1)
    d = pltpu.async_copy(msg_hbm.at[pl.ds(base, CH)], bufs[0], sems[0])
    for c in range(NCH):
        if c + 1 < NCH:
            d_next = pltpu.async_copy(
                msg_hbm.at[pl.ds(base + (c + 1) * CH, CH)],
                bufs[(c + 1) % 2], sems[(c + 1) % 2])
        d.wait()
        pltpu.sync_copy(bufs[c % 2], acc_sh.at[idx_v.at[c]], add=True)
        if c + 1 < NCH:
            d = d_next
    plsc.subcore_barrier()

    pltpu.sync_copy(acc_sh.at[pl.ds(sid * RPT, RPT)],
                    out_hbm.at[pl.ds(cid * NP + sid * RPT, RPT)])


def _k4(msg, dst_r):
    mesh = plsc.VectorSubcoreMesh(core_axis_name="c", subcore_axis_name="s", num_cores=2, num_subcores=16)
    f = functools.partial(
        pl.kernel,
        out_type=jax.ShapeDtypeStruct((2 * NP, H), F32),
        mesh=mesh,
        scratch_types=[
            pltpu.VMEM((NCH, CH), jnp.int32),
            pltpu.VMEM((CH, H), F32),
            pltpu.VMEM((CH, H), F32),
            pltpu.VMEM((ZR, H), F32),
            pltpu.VMEM_SHARED((NP, H), F32),
            pltpu.SemaphoreType.DMA,
            pltpu.SemaphoreType.DMA,
        ],
    )(_k4_body)
    return f(msg, dst_r)


def _k5_body(h0_ref, p0_ref, p1_ref, root_ref, cb_ref,
             wir_ref, wiz_ref, win_ref, whr_ref, whz_ref, whn_ref,
             bir_ref, biz_ref, bin_ref, bhr_ref, bhz_ref, bhn_ref,
             mw1_ref, mb1_ref, mw2_ref, mb2_ref, o_ref):
    xt = h0_ref[...]
    aggr = p0_ref[...] + p1_ref[...]
    conv = _silu(aggr + jnp.dot(xt, root_ref[...],
                                preferred_element_type=F32) + cb_ref[...])
    gir = jnp.dot(conv, wir_ref[...], preferred_element_type=F32) + bir_ref[...]
    giz = jnp.dot(conv, wiz_ref[...], preferred_element_type=F32) + biz_ref[...]
    gin = jnp.dot(conv, win_ref[...], preferred_element_type=F32) + bin_ref[...]
    h = xt
    for _ in range(L):
        ghr = jnp.dot(h, whr_ref[...], preferred_element_type=F32) + bhr_ref[...]
        ghz = jnp.dot(h, whz_ref[...], preferred_element_type=F32) + bhz_ref[...]
        ghn = jnp.dot(h, whn_ref[...], preferred_element_type=F32) + bhn_ref[...]
        r = jax.nn.sigmoid(gir + ghr)
        z = jax.nn.sigmoid(giz + ghz)
        n = jnp.tanh(gin + r * ghn)
        h = (1.0 - z) * n + z * h
    mixed = jnp.dot(_silu(jnp.dot(h, mw1_ref[...],
                                  preferred_element_type=F32) + mb1_ref[...]),
                    mw2_ref[...], preferred_element_type=F32) + mb2_ref[...]
    o_ref[...] = mixed


def _k5(h0, p0, p1, weights):
    blocked = pl.BlockSpec((NB, H), lambda i: (i, 0))
    wspec = [pl.BlockSpec(w.shape, lambda i: (0, 0)) for w in weights]
    return pl.pallas_call(
        _k5_body,
        grid=(N // NB,),
        in_specs=[blocked, blocked, blocked] + wspec,
        out_specs=blocked,
        out_shape=jax.ShapeDtypeStruct((N, H), F32),
    )(h0, p0, p1, *weights)


def _k6_body(mx_ref, bc_ref, br_ref, lwi_ref, lwh_ref, lb_ref,
             ow1_ref, ob1_ref, ow2_ref, ob2_ref, o_ref, e_scr, a_scr):
    CN = N // NB
    hl = jnp.zeros((B, H), F32)
    cl = jnp.zeros((B, H), F32)
    q_star = jnp.zeros((B, 2 * H), F32)
    neg = jnp.float32(-1e30)

    for _ in range(M):
        g = (jnp.dot(q_star, lwi_ref[...], preferred_element_type=F32)
             + jnp.dot(hl, lwh_ref[...], preferred_element_type=F32)
             + lb_ref[...])
        ig = jax.nn.sigmoid(g[:, :H])
        fg = jax.nn.sigmoid(g[:, H:2 * H])
        gg = jnp.tanh(g[:, 2 * H:3 * H])
        og = jax.nn.sigmoid(g[:, 3 * H:])
        cl = fg * cl + ig * gg
        hl = og * jnp.tanh(cl)
        q = hl

        em = jnp.full((1, B), neg, F32)
        for c in range(CN):
            mxc = mx_ref[pl.ds(c * NB, NB), :]
            bcc = bc_ref[pl.ds(c * NB, NB), :]
            oh = (lax.broadcasted_iota(jnp.int32, (NB, B), 1) == bcc)
            ohf = oh.astype(F32)
            qb = jnp.dot(ohf, q, preferred_element_type=F32)
            e_c = jnp.sum(mxc * qb, axis=1, keepdims=True)
            e_scr[pl.ds(c * NB, NB), :] = e_c
            em = jnp.maximum(em, jnp.max(jnp.where(oh, e_c, neg), axis=0,
                                         keepdims=True))

        den = jnp.zeros((1, B), F32)
        for c in range(CN):
            bcc = bc_ref[pl.ds(c * NB, NB), :]
            oh = (lax.broadcasted_iota(jnp.int32, (NB, B), 1) == bcc)
            ohf = oh.astype(F32)
            e_c = e_scr[pl.ds(c * NB, NB), :]
            emg = jnp.sum(ohf * em, axis=1, keepdims=True)
            a_c = jnp.exp(e_c - emg)
            a_scr[pl.ds(c * NB, NB), :] = a_c
            den = den + jnp.sum(ohf * a_c, axis=0, keepdims=True)

        r2 = jnp.zeros((B, H), F32)
        for c in range(CN):
            mxc = mx_ref[pl.ds(c * NB, NB), :]
            bcc = bc_ref[pl.ds(c * NB, NB), :]
            brc = br_ref[c]
            oh = (lax.broadcasted_iota(jnp.int32, (NB, B), 1) == bcc)
            ohf = oh.astype(F32)
            ohT = (lax.broadcasted_iota(jnp.int32, (B, NB), 0) == brc)
            a_c = a_scr[pl.ds(c * NB, NB), :]
            deng = jnp.sum(ohf * den, axis=1, keepdims=True)
            an = a_c / deng
            r2 = r2 + jnp.dot(ohT.astype(F32), an * mxc,
                              preferred_element_type=F32)
        q_star = jnp.concatenate([q, r2], axis=1)

    o = (jnp.dot(_silu(jnp.dot(q_star, ow1_ref[...],
                               preferred_element_type=F32) + ob1_ref[...]),
                 ow2_ref[...], preferred_element_type=F32) + ob2_ref[...])
    o_ref[...] = o


def _k6(mixed, batch_col, batch_row, lwi, lwh, lbr, ow1, ob1, ow2, ob2):
    return pl.pallas_call(
        _k6_body,
        out_shape=jax.ShapeDtypeStruct((B, 1), F32),
        scratch_shapes=[
            pltpu.VMEM((N, 1), F32),
            pltpu.VMEM((N, 1), F32),
        ],
    )(mixed, batch_col, batch_row, lwi, lwh, lbr, ow1, ob1, ow2, ob2)


def kernel(x, edge_attr, W1, b1, nnA1, nnb1, nnA2, nnb2, root, conv_b,
           gru_Wih, gru_Whh, gru_bih, gru_bhh, mix_W1, mix_b1, mix_W2,
           mix_b2, lstm_Wih, lstm_Whh, lstm_bih, lstm_bhh, out_W1, out_b1,
           out_W2, out_b2, edge_index, batch):
    b1r = b1.reshape(1, H)
    a1c = nnA1.transpose(1, 0, 2).reshape(ED, T * HID)
    b1c = nnb1.reshape(1, T * HID)
    nnA2p = nnA2.reshape(T, HID, TD, TD).transpose(0, 1, 3, 2).reshape(
        T, HID, TD * TD)
    nnb2p = nnb2.reshape(T, TD, TD).transpose(0, 2, 1).reshape(
        T, 1, TD * TD)
    red = jnp.kron(jnp.eye(TD, dtype=F32), jnp.ones((TD, 1), F32))

    rootbd = block_diag(*[root[t] for t in range(T)])
    cbr = conv_b.reshape(1, H)
    wir = block_diag(*[gru_Wih[t][:, :TD] for t in range(T)])
    wiz = block_diag(*[gru_Wih[t][:, TD:2 * TD] for t in range(T)])
    win = block_diag(*[gru_Wih[t][:, 2 * TD:] for t in range(T)])
    whr = block_diag(*[gru_Whh[t][:, :TD] for t in range(T)])
    whz = block_diag(*[gru_Whh[t][:, TD:2 * TD] for t in range(T)])
    whn = block_diag(*[gru_Whh[t][:, 2 * TD:] for t in range(T)])
    bir = gru_bih[:, :TD].reshape(1, H)
    biz = gru_bih[:, TD:2 * TD].reshape(1, H)
    bin_ = gru_bih[:, 2 * TD:].reshape(1, H)
    bhr = gru_bhh[:, :TD].reshape(1, H)
    bhz = gru_bhh[:, TD:2 * TD].reshape(1, H)
    bhn = gru_bhh[:, 2 * TD:].reshape(1, H)
    mb1 = mix_b1.reshape(1, H)
    mb2 = mix_b2.reshape(1, H)

    lbr = (lstm_bih + lstm_bhh).reshape(1, 4 * H)
    ob1 = out_b1.reshape(1, H)
    ob2 = out_b2.reshape(1, 1)

    pad = EP - E
    src_r = jnp.concatenate(
        [edge_index[0], jnp.zeros((pad,), jnp.int32)]).reshape(NW, NCH, CH)
    dst_r = jnp.concatenate(
        [edge_index[1], jnp.full((pad,), NP - 1, jnp.int32)]).reshape(
            NW, NCH, CH)
    ea_pad = jnp.concatenate(
        [edge_attr, jnp.zeros((pad, ED), F32)], axis=0)
    batch_col = batch.reshape(N, 1)
    batch_row = batch.reshape(N // NB, 1, NB)

    h0 = _k1(x, W1, b1r)
    xj = _k2(h0, src_r)
    msg = _k3(ea_pad, xj, a1c, b1c, nnA2p, nnb2p, red)
    parts = _k4(msg, dst_r)
    k5w = [rootbd, cbr, wir, wiz, win, whr, whz, whn,
           bir, biz, bin_, bhr, bhz, bhn, mix_W1, mb1, mix_W2, mb2]
    mixed = _k5(h0, parts[:N], parts[NP:NP + N], k5w)
    o = _k6(mixed, batch_col, batch_row, lstm_Wih, lstm_Whh, lbr,
            out_W1, ob1, out_W2, ob2)
    return o[:, 0]
